# baseline (speedup 1.0000x reference)
import functools

import jax
import jax.numpy as jnp
from jax import lax
from jax.experimental import pallas as pl
from jax.experimental.pallas import tpu as pltpu
from jax.experimental.pallas import tpu_sc as plsc

N = 10000
E = 160000
D_IN = 256
D_HID = 256
D_OUT = 64

NC = 2
NS = 16
LANES = 16

CH = 125
N_CHUNKS = E // CH
CP_WORKERS = 10
CP_ROWS = N // CP_WORKERS
ZROWS = 40

HR = 80
HC = 128

_mesh = lambda: plsc.VectorSubcoreMesh(core_axis_name="c", subcore_axis_name="s")


def _fill_const(ref, nrows, ncols, val):
    v = jnp.full((LANES,), val, jnp.float32)

    def row(r, _):
        for k in range(ncols // LANES):
            ref[r, pl.ds(k * LANES, LANES)] = v
        return 0

    lax.fori_loop(0, nrows, row, 0)


def _fill_zeros(ref, nrows, ncols):
    _fill_const(ref, nrows, ncols, 0.0)


DW = 128


def _make_degrees():
    chunks_per_sub = N_CHUNKS // NS

    @functools.partial(
        pl.kernel,
        out_type=jax.ShapeDtypeStruct((NC, N, DW), jnp.float32),
        mesh=_mesh(),
        scratch_types=[
            pltpu.VMEM_SHARED((N, DW), jnp.float32),
            pltpu.VMEM((2, 1, CH), jnp.int32),
            pltpu.VMEM((CH, DW), jnp.float32),
            [pltpu.SemaphoreType.DMA] * 2,
        ],
    )
    def degrees(ei_hbm, dummy_hbm, ones_hbm, zeros_hbm, out_hbm, acc, ib, ones, sems):
        c = lax.axis_index("c")
        s = lax.axis_index("s")

        pltpu.sync_copy(ones_hbm, ones)

        @pl.when(s < CP_WORKERS)
        def _():
            def zslice(j, _):
                pltpu.sync_copy(zeros_hbm, acc.at[pl.ds(s * CP_ROWS + j * ZROWS, ZROWS)])
                return 0

            lax.fori_loop(0, CP_ROWS // ZROWS, zslice, 0)

        plsc.subcore_barrier()

        base = c * N_CHUNKS + s * chunks_per_sub

        def fire(j, b):
            pltpu.sync_copy(ei_hbm.at[base + j], ib.at[b])
            pltpu.async_copy(ones, acc.at[ib.at[b, 0]], sems[b], add=True)

        for b in range(2):
            fire(b, b)
        n_outer = chunks_per_sub // 2

        def chunk(j0, _):
            for b in range(2):
                pltpu.make_async_copy(dummy_hbm, ones, sems[b]).wait()

                @pl.when(j0 < n_outer - 1)
                def _():
                    fire(j0 * 2 + b + 2, b)

            return 0

        lax.fori_loop(0, n_outer, chunk, 0)
        plsc.subcore_barrier()

        @pl.when(s < CP_WORKERS)
        def _():
            rr = pl.ds(s * CP_ROWS, CP_ROWS)
            pltpu.sync_copy(acc.at[rr], out_hbm.at[c, rr])

    return degrees


def _make_agg(W, colsplit):
    if colsplit:
        chunks = N_CHUNKS // NS
        g_shape = (NC, N, W)
    else:
        chunks = N_CHUNKS // (NC * NS)
        g_shape = (N, W)

    NB = 2

    @functools.partial(
        pl.kernel,
        out_type=jax.ShapeDtypeStruct((NC, N, W), jnp.float32),
        mesh=_mesh(),
        scratch_types=[
            pltpu.VMEM_SHARED((N, W), jnp.float32),
            pltpu.VMEM((NB, 1, CH), jnp.int32),
            pltpu.VMEM((NB, 1, CH), jnp.int32),
            pltpu.VMEM((NB, CH, W), jnp.float32),
            [pltpu.SemaphoreType.DMA] * NB,
            [pltpu.SemaphoreType.DMA] * NB,
        ],
    )
    def agg(g_hbm, src_hbm, dst_hbm, dummy_hbm, zeros_hbm, out_hbm, acc, ixs, ixd, rows, sem_g, sem_s):
        c = lax.axis_index("c")
        s = lax.axis_index("s")

        @pl.when(s < CP_WORKERS)
        def _():
            def zslice(j, _):
                pltpu.sync_copy(zeros_hbm, acc.at[pl.ds(s * CP_ROWS + j * ZROWS, ZROWS)])
                return 0

            lax.fori_loop(0, CP_ROWS // ZROWS, zslice, 0)

        plsc.subcore_barrier()

        base = (s if colsplit else c * NS + s) * chunks
        n_outer = chunks // NB
        gsrc = g_hbm.at[c] if colsplit else g_hbm

        def fetch(j, b):
            pltpu.sync_copy(src_hbm.at[base + j], ixs.at[b])
            pltpu.sync_copy(dst_hbm.at[base + j], ixd.at[b])
            pltpu.async_copy(gsrc.at[ixs.at[b, 0]], rows.at[b], sem_g[b])

        for b in range(NB):
            fetch(b, b)

        def body(j0, _):
            for b in range(NB):
                pltpu.make_async_copy(dummy_hbm, rows.at[b], sem_g[b]).wait()
                pltpu.async_copy(rows.at[b], acc.at[ixd.at[b, 0]], sem_s[b], add=True)
                pltpu.make_async_copy(dummy_hbm, rows.at[b], sem_s[b]).wait()

                @pl.when(j0 < n_outer - 1)
                def _():
                    fetch(j0 * NB + b + NB, b)

            return 0

        lax.fori_loop(0, n_outer, body, 0)
        plsc.subcore_barrier()

        @pl.when(s < CP_WORKERS)
        def _():
            pltpu.sync_copy(
                acc.at[pl.ds(s * CP_ROWS, CP_ROWS)],
                out_hbm.at[c, pl.ds(s * CP_ROWS, CP_ROWS)],
            )

    return agg


RB = 1000


def _norms(deg_ref):
    ns = lax.rsqrt(jnp.maximum(deg_ref[0][:, 0:1], 1.0))
    nd = lax.rsqrt(jnp.maximum(deg_ref[1][:, 0:1], 1.0))
    return ns, nd


def _stage_a(deg_ref, feat_ref, g0_ref):
    ns, _ = _norms(deg_ref)
    g = feat_ref[...] * ns
    g0_ref[0] = g[:, :128]
    g0_ref[1] = g[:, 128:]


def _stage_bc(deg_ref, agg_ref, w_ref, b_ref, w2_ref, out_ref, *, last):
    ns, nd = _norms(deg_ref)
    m = jnp.dot(agg_ref[0], w_ref[:128, :], preferred_element_type=jnp.float32)
    m = m + jnp.dot(agg_ref[1], w_ref[128:, :], preferred_element_type=jnp.float32)
    h = jnp.maximum(m * nd + b_ref[...], 0.0) * ns
    if last:
        out_ref[...] = jnp.dot(h, w2_ref[...], preferred_element_type=jnp.float32)
    else:
        out_ref[0] = h[:, :128]
        out_ref[1] = h[:, 128:]


def _stage_d(deg_ref, p_ref, b_ref, out_ref):
    _, nd = _norms(deg_ref)
    a = p_ref[0] + p_ref[1]
    out_ref[...] = a[:, :D_OUT] * nd + b_ref[...]


_DEG_SPEC = pl.BlockSpec((NC, RB, DW), lambda i: (0, i, 0))


def _tc_stage_a(deg, features):
    return pl.pallas_call(
        _stage_a,
        grid=(N // RB,),
        in_specs=[_DEG_SPEC, pl.BlockSpec((RB, D_IN), lambda i: (i, 0))],
        out_specs=pl.BlockSpec((NC, RB, 128), lambda i: (0, i, 0)),
        out_shape=jax.ShapeDtypeStruct((NC, N, 128), jnp.float32),
    )(deg, features)


def _tc_stage_b(deg, agg, w, b):
    def wrapped(deg_ref, agg_ref, w_ref, b_ref, out_ref):
        _stage_bc(deg_ref, agg_ref, w_ref, b_ref, None, out_ref, last=False)

    return pl.pallas_call(
        wrapped,
        grid=(N // RB,),
        in_specs=[
            _DEG_SPEC,
            pl.BlockSpec((NC, RB, 128), lambda i: (0, i, 0)),
            pl.BlockSpec((D_HID, D_HID), lambda i: (0, 0)),
            pl.BlockSpec((1, D_HID), lambda i: (0, 0)),
        ],
        out_specs=pl.BlockSpec((NC, RB, 128), lambda i: (0, i, 0)),
        out_shape=jax.ShapeDtypeStruct((NC, N, 128), jnp.float32),
    )(deg, agg, w, b)


def _tc_stage_c(deg, agg, w, b, w2):
    body = functools.partial(_stage_bc, last=True)
    return pl.pallas_call(
        body,
        grid=(N // RB,),
        in_specs=[
            _DEG_SPEC,
            pl.BlockSpec((NC, RB, 128), lambda i: (0, i, 0)),
            pl.BlockSpec((D_HID, D_HID), lambda i: (0, 0)),
            pl.BlockSpec((1, D_HID), lambda i: (0, 0)),
            pl.BlockSpec((D_HID, 128), lambda i: (0, 0)),
        ],
        out_specs=pl.BlockSpec((RB, 128), lambda i: (i, 0)),
        out_shape=jax.ShapeDtypeStruct((N, 128), jnp.float32),
    )(deg, agg, w, b, w2)


def _tc_stage_d(deg, p, b2):
    return pl.pallas_call(
        _stage_d,
        grid=(N // RB,),
        in_specs=[
            _DEG_SPEC,
            pl.BlockSpec((NC, RB, 128), lambda i: (0, i, 0)),
            pl.BlockSpec((1, D_OUT), lambda i: (0, 0)),
        ],
        out_specs=pl.BlockSpec((RB, D_OUT), lambda i: (i, 0)),
        out_shape=jax.ShapeDtypeStruct((N, D_OUT), jnp.float32),
    )(deg, p, b2)


@jax.jit
def kernel(features, edge_index, W0, b0, W1, b1, W2, b2):
    src3d = edge_index[0].reshape(N_CHUNKS, 1, CH)
    dst3d = edge_index[1].reshape(N_CHUNKS, 1, CH)
    ei3d = edge_index.reshape(2 * N_CHUNKS, 1, CH)

    dummy = jnp.zeros((CH, 128), jnp.float32)
    zeros = jnp.zeros((ZROWS, 128), jnp.float32)
    ones = jnp.ones((CH, 128), jnp.float32)

    deg = _make_degrees()(ei3d, dummy, ones, zeros)
    g0 = _tc_stage_a(deg, features)
    a0 = _make_agg(128, True)(g0, src3d, dst3d, dummy, zeros)
    g1 = _tc_stage_b(deg, a0, W0, b0.reshape(1, D_HID))
    a1 = _make_agg(128, True)(g1, src3d, dst3d, dummy, zeros)
    w2p = jnp.concatenate(
        [W2, jnp.zeros((D_HID, 128 - D_OUT), jnp.float32)], axis=1
    )
    g2 = _tc_stage_c(deg, a1, W1, b1.reshape(1, D_HID), w2p)
    p2 = _make_agg(128, False)(g2, src3d, dst3d, dummy, zeros)
    out = _tc_stage_d(deg, p2, b2.reshape(1, D_OUT))
    return out


# speedup vs baseline: 1.1511x; 1.1511x over previous
import functools

import jax
import jax.numpy as jnp
from jax import lax
from jax.experimental import pallas as pl
from jax.experimental.pallas import tpu as pltpu
from jax.experimental.pallas import tpu_sc as plsc

N = 10000
E = 160000
D_IN = 256
D_HID = 256
D_OUT = 64

NC = 2
NS = 16
LANES = 16

CH = 125
N_CHUNKS = E // CH
CP_WORKERS = 10
CP_ROWS = N // CP_WORKERS
ZROWS = 40

HR = 80
HC = 128

_mesh = lambda: plsc.VectorSubcoreMesh(core_axis_name="c", subcore_axis_name="s")


def _fill_const(ref, nrows, ncols, val):
    v = jnp.full((LANES,), val, jnp.float32)

    def row(r, _):
        for k in range(ncols // LANES):
            ref[r, pl.ds(k * LANES, LANES)] = v
        return 0

    lax.fori_loop(0, nrows, row, 0)


def _fill_zeros(ref, nrows, ncols):
    _fill_const(ref, nrows, ncols, 0.0)


DW = 128


def _make_degrees():
    chunks_per_sub = N_CHUNKS // NS

    @functools.partial(
        pl.kernel,
        out_type=jax.ShapeDtypeStruct((NC, N, DW), jnp.float32),
        mesh=_mesh(),
        scratch_types=[
            pltpu.VMEM_SHARED((N, DW), jnp.float32),
            pltpu.VMEM((2, 1, CH), jnp.int32),
            pltpu.VMEM((CH, DW), jnp.float32),
            pltpu.VMEM((ZROWS, DW), jnp.float32),
            [pltpu.SemaphoreType.DMA] * 2,
        ],
    )
    def degrees(ei_hbm, dummy_hbm, out_hbm, acc, ib, ones, zbuf, sems):
        c = lax.axis_index("c")
        s = lax.axis_index("s")

        _fill_const(ones, CH, DW, 1.0)
        _fill_zeros(zbuf, ZROWS, DW)

        @pl.when(s < CP_WORKERS)
        def _():
            def zslice(j, _):
                pltpu.sync_copy(zbuf, acc.at[pl.ds(s * CP_ROWS + j * ZROWS, ZROWS)])
                return 0

            lax.fori_loop(0, CP_ROWS // ZROWS, zslice, 0)

        plsc.subcore_barrier()

        base = c * N_CHUNKS + s * chunks_per_sub

        def fire(j, b):
            pltpu.sync_copy(ei_hbm.at[base + j], ib.at[b])
            pltpu.async_copy(ones, acc.at[ib.at[b, 0]], sems[b], add=True)

        for b in range(2):
            fire(b, b)
        n_outer = chunks_per_sub // 2

        def chunk(j0, _):
            for b in range(2):
                pltpu.make_async_copy(dummy_hbm, ones, sems[b]).wait()

                @pl.when(j0 < n_outer - 1)
                def _():
                    fire(j0 * 2 + b + 2, b)

            return 0

        lax.fori_loop(0, n_outer, chunk, 0)
        plsc.subcore_barrier()

        @pl.when(s < CP_WORKERS)
        def _():
            rr = pl.ds(s * CP_ROWS, CP_ROWS)
            pltpu.sync_copy(acc.at[rr], out_hbm.at[c, rr])

    return degrees


def _make_agg(W, colsplit):
    if colsplit:
        chunks = N_CHUNKS // NS
        g_shape = (NC, N, W)
    else:
        chunks = N_CHUNKS // (NC * NS)
        g_shape = (N, W)

    NB = 2

    @functools.partial(
        pl.kernel,
        out_type=jax.ShapeDtypeStruct((NC, N, W), jnp.float32),
        mesh=_mesh(),
        scratch_types=[
            pltpu.VMEM_SHARED((N, W), jnp.float32),
            pltpu.VMEM((NB, 1, CH), jnp.int32),
            pltpu.VMEM((NB, 1, CH), jnp.int32),
            pltpu.VMEM((NB, CH, W), jnp.float32),
            pltpu.VMEM((ZROWS, W), jnp.float32),
            [pltpu.SemaphoreType.DMA] * NB,
            [pltpu.SemaphoreType.DMA] * NB,
        ],
    )
    def agg(g_hbm, src_hbm, dst_hbm, dummy_hbm, out_hbm, acc, ixs, ixd, rows, zbuf, sem_g, sem_s):
        c = lax.axis_index("c")
        s = lax.axis_index("s")

        _fill_zeros(zbuf, ZROWS, W)

        @pl.when(s < CP_WORKERS)
        def _():
            def zslice(j, _):
                pltpu.sync_copy(zbuf, acc.at[pl.ds(s * CP_ROWS + j * ZROWS, ZROWS)])
                return 0

            lax.fori_loop(0, CP_ROWS // ZROWS, zslice, 0)

        plsc.subcore_barrier()

        base = (s if colsplit else c * NS + s) * chunks
        n_outer = chunks // NB
        gsrc = g_hbm.at[c] if colsplit else g_hbm

        def fetch(j, b):
            pltpu.sync_copy(src_hbm.at[base + j], ixs.at[b])
            pltpu.sync_copy(dst_hbm.at[base + j], ixd.at[b])
            pltpu.async_copy(gsrc.at[ixs.at[b, 0]], rows.at[b], sem_g[b])

        for b in range(NB):
            fetch(b, b)

        def body(j0, _):
            for b in range(NB):
                pltpu.make_async_copy(dummy_hbm, rows.at[b], sem_g[b]).wait()
                pltpu.async_copy(rows.at[b], acc.at[ixd.at[b, 0]], sem_s[b], add=True)
                pltpu.make_async_copy(dummy_hbm, rows.at[b], sem_s[b]).wait()

                @pl.when(j0 < n_outer - 1)
                def _():
                    fetch(j0 * NB + b + NB, b)

            return 0

        lax.fori_loop(0, n_outer, body, 0)
        plsc.subcore_barrier()

        @pl.when(s < CP_WORKERS)
        def _():
            pltpu.sync_copy(
                acc.at[pl.ds(s * CP_ROWS, CP_ROWS)],
                out_hbm.at[c, pl.ds(s * CP_ROWS, CP_ROWS)],
            )

    return agg


RB = 1000


def _norms(deg_ref):
    ns = lax.rsqrt(jnp.maximum(deg_ref[0][:, 0:1], 1.0))
    nd = lax.rsqrt(jnp.maximum(deg_ref[1][:, 0:1], 1.0))
    return ns, nd


def _stage_a(deg_ref, feat_ref, g0_ref):
    ns, _ = _norms(deg_ref)
    g = feat_ref[...] * ns
    g0_ref[0] = g[:, :128]
    g0_ref[1] = g[:, 128:]


def _stage_bc(deg_ref, agg_ref, w_ref, b_ref, w2_ref, out_ref, *, last):
    ns, nd = _norms(deg_ref)
    m = jnp.dot(agg_ref[0], w_ref[:128, :], preferred_element_type=jnp.float32)
    m = m + jnp.dot(agg_ref[1], w_ref[128:, :], preferred_element_type=jnp.float32)
    h = jnp.maximum(m * nd + b_ref[...], 0.0) * ns
    if last:
        out_ref[...] = jnp.dot(h, w2_ref[...], preferred_element_type=jnp.float32)
    else:
        out_ref[0] = h[:, :128]
        out_ref[1] = h[:, 128:]


def _stage_d(deg_ref, p_ref, b_ref, out_ref):
    _, nd = _norms(deg_ref)
    a = p_ref[0] + p_ref[1]
    out_ref[...] = a[:, :D_OUT] * nd + b_ref[...]


_DEG_SPEC = pl.BlockSpec((NC, RB, DW), lambda i: (0, i, 0))


def _tc_stage_a(deg, features):
    return pl.pallas_call(
        _stage_a,
        grid=(N // RB,),
        in_specs=[_DEG_SPEC, pl.BlockSpec((RB, D_IN), lambda i: (i, 0))],
        out_specs=pl.BlockSpec((NC, RB, 128), lambda i: (0, i, 0)),
        out_shape=jax.ShapeDtypeStruct((NC, N, 128), jnp.float32),
    )(deg, features)


def _tc_stage_b(deg, agg, w, b):
    def wrapped(deg_ref, agg_ref, w_ref, b_ref, out_ref):
        _stage_bc(deg_ref, agg_ref, w_ref, b_ref, None, out_ref, last=False)

    return pl.pallas_call(
        wrapped,
        grid=(N // RB,),
        in_specs=[
            _DEG_SPEC,
            pl.BlockSpec((NC, RB, 128), lambda i: (0, i, 0)),
            pl.BlockSpec((D_HID, D_HID), lambda i: (0, 0)),
            pl.BlockSpec((1, D_HID), lambda i: (0, 0)),
        ],
        out_specs=pl.BlockSpec((NC, RB, 128), lambda i: (0, i, 0)),
        out_shape=jax.ShapeDtypeStruct((NC, N, 128), jnp.float32),
    )(deg, agg, w, b)


def _tc_stage_c(deg, agg, w, b, w2):
    body = functools.partial(_stage_bc, last=True)
    return pl.pallas_call(
        body,
        grid=(N // RB,),
        in_specs=[
            _DEG_SPEC,
            pl.BlockSpec((NC, RB, 128), lambda i: (0, i, 0)),
            pl.BlockSpec((D_HID, D_HID), lambda i: (0, 0)),
            pl.BlockSpec((1, D_HID), lambda i: (0, 0)),
            pl.BlockSpec((D_HID, 128), lambda i: (0, 0)),
        ],
        out_specs=pl.BlockSpec((RB, 128), lambda i: (i, 0)),
        out_shape=jax.ShapeDtypeStruct((N, 128), jnp.float32),
    )(deg, agg, w, b, w2)


def _tc_stage_d(deg, p, b2):
    return pl.pallas_call(
        _stage_d,
        grid=(N // RB,),
        in_specs=[
            _DEG_SPEC,
            pl.BlockSpec((NC, RB, 128), lambda i: (0, i, 0)),
            pl.BlockSpec((1, D_OUT), lambda i: (0, 0)),
        ],
        out_specs=pl.BlockSpec((RB, D_OUT), lambda i: (i, 0)),
        out_shape=jax.ShapeDtypeStruct((N, D_OUT), jnp.float32),
    )(deg, p, b2)


@jax.jit
def kernel(features, edge_index, W0, b0, W1, b1, W2, b2):
    src3d = edge_index[0].reshape(N_CHUNKS, 1, CH)
    dst3d = edge_index[1].reshape(N_CHUNKS, 1, CH)
    ei3d = edge_index.reshape(2 * N_CHUNKS, 1, CH)

    dummy = jnp.zeros((CH, 128), jnp.float32)

    deg = _make_degrees()(ei3d, dummy)
    g0 = _tc_stage_a(deg, features)
    a0 = _make_agg(128, True)(g0, src3d, dst3d, dummy)
    g1 = _tc_stage_b(deg, a0, W0, b0.reshape(1, D_HID))
    a1 = _make_agg(128, True)(g1, src3d, dst3d, dummy)
    w2p = jnp.concatenate(
        [W2, jnp.zeros((D_HID, 128 - D_OUT), jnp.float32)], axis=1
    )
    g2 = _tc_stage_c(deg, a1, W1, b1.reshape(1, D_HID), w2p)
    p2 = _make_agg(128, False)(g2, src3d, dst3d, dummy)
    out = _tc_stage_d(deg, p2, b2.reshape(1, D_OUT))
    return out


# speedup vs baseline: 1.1522x; 1.0010x over previous
import functools

import jax
import jax.numpy as jnp
from jax import lax
from jax.experimental import pallas as pl
from jax.experimental.pallas import tpu as pltpu
from jax.experimental.pallas import tpu_sc as plsc

N = 10000
E = 160000
D_IN = 256
D_HID = 256
D_OUT = 64

NC = 2
NS = 16
LANES = 16

CH = 125
N_CHUNKS = E // CH
CP_WORKERS = 10
CP_ROWS = N // CP_WORKERS
ZROWS = 40

HR = 80
HC = 128

_mesh = lambda: plsc.VectorSubcoreMesh(core_axis_name="c", subcore_axis_name="s")


def _fill_const(ref, nrows, ncols, val):
    v = jnp.full((LANES,), val, jnp.float32)

    def row(r, _):
        for k in range(ncols // LANES):
            ref[r, pl.ds(k * LANES, LANES)] = v
        return 0

    lax.fori_loop(0, nrows, row, 0)


def _fill_zeros(ref, nrows, ncols):
    _fill_const(ref, nrows, ncols, 0.0)


DW = 128


def _make_degrees():
    chunks_per_sub = N_CHUNKS // NS

    @functools.partial(
        pl.kernel,
        out_type=jax.ShapeDtypeStruct((NC, N, DW), jnp.float32),
        mesh=_mesh(),
        scratch_types=[
            pltpu.VMEM_SHARED((N, DW), jnp.float32),
            pltpu.VMEM((2, 1, CH), jnp.int32),
            pltpu.VMEM((CH, DW), jnp.float32),
            pltpu.VMEM((ZROWS, DW), jnp.float32),
            [pltpu.SemaphoreType.DMA] * 2,
        ],
    )
    def degrees(ei_hbm, dummy_hbm, out_hbm, acc, ib, ones, zbuf, sems):
        c = lax.axis_index("c")
        s = lax.axis_index("s")

        _fill_const(ones, CH, DW, 1.0)
        _fill_zeros(zbuf, ZROWS, DW)

        @pl.when(s < CP_WORKERS)
        def _():
            def zslice(j, _):
                pltpu.sync_copy(zbuf, acc.at[pl.ds(s * CP_ROWS + j * ZROWS, ZROWS)])
                return 0

            lax.fori_loop(0, CP_ROWS // ZROWS, zslice, 0)

        plsc.subcore_barrier()

        base = c * N_CHUNKS + s * chunks_per_sub

        def fire(j, b):
            pltpu.sync_copy(ei_hbm.at[base + j], ib.at[b])
            pltpu.async_copy(ones, acc.at[ib.at[b, 0]], sems[b], add=True)

        for b in range(2):
            fire(b, b)
        n_outer = chunks_per_sub // 2

        def chunk(j0, _):
            for b in range(2):
                pltpu.make_async_copy(dummy_hbm, ones, sems[b]).wait()

                @pl.when(j0 < n_outer - 1)
                def _():
                    fire(j0 * 2 + b + 2, b)

            return 0

        lax.fori_loop(0, n_outer, chunk, 0)
        plsc.subcore_barrier()

        @pl.when(s < CP_WORKERS)
        def _():
            rr = pl.ds(s * CP_ROWS, CP_ROWS)
            pltpu.sync_copy(acc.at[rr], out_hbm.at[c, rr])

    return degrees


def _make_agg(W, colsplit):
    if colsplit:
        chunks = N_CHUNKS // NS
        g_shape = (NC, N, W)
    else:
        chunks = N_CHUNKS // (NC * NS)
        g_shape = (N, W)

    NB = 2

    @functools.partial(
        pl.kernel,
        out_type=jax.ShapeDtypeStruct((NC, N, W), jnp.float32),
        mesh=_mesh(),
        scratch_types=[
            pltpu.VMEM_SHARED((N, W), jnp.float32),
            pltpu.VMEM((NB, 1, CH), jnp.int32),
            pltpu.VMEM((NB, 1, CH), jnp.int32),
            pltpu.VMEM((NB, CH, W), jnp.float32),
            pltpu.VMEM((ZROWS, W), jnp.float32),
            [pltpu.SemaphoreType.DMA] * NB,
            [pltpu.SemaphoreType.DMA] * NB,
        ],
    )
    def agg(g_hbm, src_hbm, dst_hbm, dummy_hbm, out_hbm, acc, ixs, ixd, rows, zbuf, sem_g, sem_s):
        c = lax.axis_index("c")
        s = lax.axis_index("s")

        _fill_zeros(zbuf, ZROWS, W)

        def zslice(j, _):
            pltpu.sync_copy(zbuf, acc.at[pl.ds(s * CP_ROWS + j * ZROWS, ZROWS)])
            return 0

        base = (s if colsplit else c * NS + s) * chunks
        n_outer = chunks // NB
        gsrc = g_hbm.at[c] if colsplit else g_hbm

        def fetch(j, b):
            pltpu.sync_copy(src_hbm.at[base + j], ixs.at[b])
            pltpu.sync_copy(dst_hbm.at[base + j], ixd.at[b])
            pltpu.async_copy(gsrc.at[ixs.at[b, 0]], rows.at[b], sem_g[b])

        for b in range(NB):
            fetch(b, b)

        @pl.when(s < CP_WORKERS)
        def _():
            lax.fori_loop(0, CP_ROWS // ZROWS, zslice, 0)

        plsc.subcore_barrier()

        def body(j0, _):
            for b in range(NB):
                pltpu.make_async_copy(dummy_hbm, rows.at[b], sem_g[b]).wait()
                pltpu.async_copy(rows.at[b], acc.at[ixd.at[b, 0]], sem_s[b], add=True)
                pltpu.make_async_copy(dummy_hbm, rows.at[b], sem_s[b]).wait()

                @pl.when(j0 < n_outer - 1)
                def _():
                    fetch(j0 * NB + b + NB, b)

            return 0

        lax.fori_loop(0, n_outer, body, 0)
        plsc.subcore_barrier()

        @pl.when(s < CP_WORKERS)
        def _():
            pltpu.sync_copy(
                acc.at[pl.ds(s * CP_ROWS, CP_ROWS)],
                out_hbm.at[c, pl.ds(s * CP_ROWS, CP_ROWS)],
            )

    return agg


RB = 1000


def _norms(deg_ref):
    ns = lax.rsqrt(jnp.maximum(deg_ref[0][:, 0:1], 1.0))
    nd = lax.rsqrt(jnp.maximum(deg_ref[1][:, 0:1], 1.0))
    return ns, nd


def _stage_a(deg_ref, feat_ref, g0_ref):
    ns, _ = _norms(deg_ref)
    g = feat_ref[...] * ns
    g0_ref[0] = g[:, :128]
    g0_ref[1] = g[:, 128:]


def _stage_bc(deg_ref, agg_ref, w_ref, b_ref, w2_ref, out_ref, *, last):
    ns, nd = _norms(deg_ref)
    m = jnp.dot(agg_ref[0], w_ref[:128, :], preferred_element_type=jnp.float32)
    m = m + jnp.dot(agg_ref[1], w_ref[128:, :], preferred_element_type=jnp.float32)
    h = jnp.maximum(m * nd + b_ref[...], 0.0) * ns
    if last:
        out_ref[...] = jnp.dot(h, w2_ref[...], preferred_element_type=jnp.float32)
    else:
        out_ref[0] = h[:, :128]
        out_ref[1] = h[:, 128:]


def _stage_d(deg_ref, p_ref, b_ref, out_ref):
    _, nd = _norms(deg_ref)
    a = p_ref[0] + p_ref[1]
    out_ref[...] = a[:, :D_OUT] * nd + b_ref[...]


_DEG_SPEC = pl.BlockSpec((NC, RB, DW), lambda i: (0, i, 0))


def _tc_stage_a(deg, features):
    return pl.pallas_call(
        _stage_a,
        grid=(N // RB,),
        in_specs=[_DEG_SPEC, pl.BlockSpec((RB, D_IN), lambda i: (i, 0))],
        out_specs=pl.BlockSpec((NC, RB, 128), lambda i: (0, i, 0)),
        out_shape=jax.ShapeDtypeStruct((NC, N, 128), jnp.float32),
    )(deg, features)


def _tc_stage_b(deg, agg, w, b):
    def wrapped(deg_ref, agg_ref, w_ref, b_ref, out_ref):
        _stage_bc(deg_ref, agg_ref, w_ref, b_ref, None, out_ref, last=False)

    return pl.pallas_call(
        wrapped,
        grid=(N // RB,),
        in_specs=[
            _DEG_SPEC,
            pl.BlockSpec((NC, RB, 128), lambda i: (0, i, 0)),
            pl.BlockSpec((D_HID, D_HID), lambda i: (0, 0)),
            pl.BlockSpec((1, D_HID), lambda i: (0, 0)),
        ],
        out_specs=pl.BlockSpec((NC, RB, 128), lambda i: (0, i, 0)),
        out_shape=jax.ShapeDtypeStruct((NC, N, 128), jnp.float32),
    )(deg, agg, w, b)


def _tc_stage_c(deg, agg, w, b, w2):
    body = functools.partial(_stage_bc, last=True)
    return pl.pallas_call(
        body,
        grid=(N // RB,),
        in_specs=[
            _DEG_SPEC,
            pl.BlockSpec((NC, RB, 128), lambda i: (0, i, 0)),
            pl.BlockSpec((D_HID, D_HID), lambda i: (0, 0)),
            pl.BlockSpec((1, D_HID), lambda i: (0, 0)),
            pl.BlockSpec((D_HID, 128), lambda i: (0, 0)),
        ],
        out_specs=pl.BlockSpec((RB, 128), lambda i: (i, 0)),
        out_shape=jax.ShapeDtypeStruct((N, 128), jnp.float32),
    )(deg, agg, w, b, w2)


def _tc_stage_d(deg, p, b2):
    return pl.pallas_call(
        _stage_d,
        grid=(N // RB,),
        in_specs=[
            _DEG_SPEC,
            pl.BlockSpec((NC, RB, 128), lambda i: (0, i, 0)),
            pl.BlockSpec((1, D_OUT), lambda i: (0, 0)),
        ],
        out_specs=pl.BlockSpec((RB, D_OUT), lambda i: (i, 0)),
        out_shape=jax.ShapeDtypeStruct((N, D_OUT), jnp.float32),
    )(deg, p, b2)


@jax.jit
def kernel(features, edge_index, W0, b0, W1, b1, W2, b2):
    src3d = edge_index[0].reshape(N_CHUNKS, 1, CH)
    dst3d = edge_index[1].reshape(N_CHUNKS, 1, CH)
    ei3d = edge_index.reshape(2 * N_CHUNKS, 1, CH)

    dummy = jnp.zeros((CH, 128), jnp.float32)

    deg = _make_degrees()(ei3d, dummy)
    g0 = _tc_stage_a(deg, features)
    a0 = _make_agg(128, True)(g0, src3d, dst3d, dummy)
    g1 = _tc_stage_b(deg, a0, W0, b0.reshape(1, D_HID))
    a1 = _make_agg(128, True)(g1, src3d, dst3d, dummy)
    w2p = jnp.concatenate(
        [W2, jnp.zeros((D_HID, 128 - D_OUT), jnp.float32)], axis=1
    )
    g2 = _tc_stage_c(deg, a1, W1, b1.reshape(1, D_HID), w2p)
    p2 = _make_agg(128, False)(g2, src3d, dst3d, dummy)
    out = _tc_stage_d(deg, p2, b2.reshape(1, D_OUT))
    return out


# speedup vs baseline: 1.1531x; 1.0008x over previous
import functools

import jax
import jax.numpy as jnp
from jax import lax
from jax.experimental import pallas as pl
from jax.experimental.pallas import tpu as pltpu
from jax.experimental.pallas import tpu_sc as plsc

N = 10000
E = 160000
D_IN = 256
D_HID = 256
D_OUT = 64

NC = 2
NS = 16
LANES = 16

CH = 125
N_CHUNKS = E // CH
CP_WORKERS = 10
CP_ROWS = N // CP_WORKERS
ZROWS = 40

_mesh = lambda: plsc.VectorSubcoreMesh(core_axis_name="c", subcore_axis_name="s")


def _fill_const(ref, nrows, ncols, val):
    v = jnp.full((LANES,), val, jnp.float32)

    def row(r, _):
        for k in range(ncols // LANES):
            ref[r, pl.ds(k * LANES, LANES)] = v
        return 0

    lax.fori_loop(0, nrows, row, 0)


def _fill_zeros(ref, nrows, ncols):
    _fill_const(ref, nrows, ncols, 0.0)


DW = 128


def _make_degrees():
    chunks_per_sub = N_CHUNKS // NS

    @functools.partial(
        pl.kernel,
        out_type=jax.ShapeDtypeStruct((NC, N, DW), jnp.float32),
        mesh=_mesh(),
        scratch_types=[
            pltpu.VMEM_SHARED((N, DW), jnp.float32),
            pltpu.VMEM((2, 1, CH), jnp.int32),
            pltpu.VMEM((CH, DW), jnp.float32),
            pltpu.VMEM((ZROWS, DW), jnp.float32),
            [pltpu.SemaphoreType.DMA] * 2,
        ],
    )
    def degrees(ei_hbm, dummy_hbm, out_hbm, acc, ib, ones, zbuf, sems):
        c = lax.axis_index("c")
        s = lax.axis_index("s")

        _fill_const(ones, CH, DW, 1.0)
        _fill_zeros(zbuf, ZROWS, DW)

        @pl.when(s < CP_WORKERS)
        def _():
            def zslice(j, _):
                pltpu.sync_copy(zbuf, acc.at[pl.ds(s * CP_ROWS + j * ZROWS, ZROWS)])
                return 0

            lax.fori_loop(0, CP_ROWS // ZROWS, zslice, 0)

        plsc.subcore_barrier()

        base = c * N_CHUNKS + s * chunks_per_sub

        def fire(j, b):
            pltpu.sync_copy(ei_hbm.at[base + j], ib.at[b])
            pltpu.async_copy(ones, acc.at[ib.at[b, 0]], sems[b], add=True)

        for b in range(2):
            fire(b, b)
        n_outer = chunks_per_sub // 2

        def chunk(j0, _):
            for b in range(2):
                pltpu.make_async_copy(dummy_hbm, ones, sems[b]).wait()

                @pl.when(j0 < n_outer - 1)
                def _():
                    fire(j0 * 2 + b + 2, b)

            return 0

        lax.fori_loop(0, n_outer, chunk, 0)
        plsc.subcore_barrier()

        @pl.when(s < CP_WORKERS)
        def _():
            rr = pl.ds(s * CP_ROWS, CP_ROWS)
            pltpu.sync_copy(acc.at[rr], out_hbm.at[c, rr])

    return degrees


def _make_agg(W, colsplit):
    if colsplit:
        chunks = N_CHUNKS // NS
    else:
        chunks = N_CHUNKS // (NC * NS)

    NB = 2

    @functools.partial(
        pl.kernel,
        out_type=jax.ShapeDtypeStruct((NC, N, W), jnp.float32),
        mesh=_mesh(),
        scratch_types=[
            pltpu.VMEM_SHARED((N, W), jnp.float32),
            pltpu.VMEM((NB, 1, CH), jnp.int32),
            pltpu.VMEM((NB, 1, CH), jnp.int32),
            pltpu.VMEM((NB, CH, W), jnp.float32),
            pltpu.VMEM((ZROWS, W), jnp.float32),
            [pltpu.SemaphoreType.DMA] * NB,
            [pltpu.SemaphoreType.DMA] * NB,
        ],
    )
    def agg(g_hbm, src_hbm, dst_hbm, dummy_hbm, out_hbm, acc, ixs, ixd, rows, zbuf, sem_g, sem_s):
        c = lax.axis_index("c")
        s = lax.axis_index("s")

        _fill_zeros(zbuf, ZROWS, W)

        def zslice(j, _):
            pltpu.sync_copy(zbuf, acc.at[pl.ds(s * CP_ROWS + j * ZROWS, ZROWS)])
            return 0

        base = (s if colsplit else c * NS + s) * chunks
        n_outer = chunks // NB
        gsrc = g_hbm.at[c] if colsplit else g_hbm

        def fetch(j, b):
            pltpu.sync_copy(src_hbm.at[base + j], ixs.at[b])
            pltpu.sync_copy(dst_hbm.at[base + j], ixd.at[b])
            pltpu.async_copy(gsrc.at[ixs.at[b, 0]], rows.at[b], sem_g[b])

        for b in range(NB):
            fetch(b, b)

        @pl.when(s < CP_WORKERS)
        def _():
            lax.fori_loop(0, CP_ROWS // ZROWS, zslice, 0)

        plsc.subcore_barrier()

        def body(j0, _):
            for b in range(NB):
                pltpu.make_async_copy(dummy_hbm, rows.at[b], sem_g[b]).wait()
                pltpu.async_copy(rows.at[b], acc.at[ixd.at[b, 0]], sem_s[b], add=True)
                pltpu.make_async_copy(dummy_hbm, rows.at[b], sem_s[b]).wait()

                @pl.when(j0 < n_outer - 1)
                def _():
                    fetch(j0 * NB + b + NB, b)

            return 0

        lax.fori_loop(0, n_outer, body, 0)
        plsc.subcore_barrier()

        @pl.when(s < CP_WORKERS)
        def _():
            pltpu.sync_copy(
                acc.at[pl.ds(s * CP_ROWS, CP_ROWS)],
                out_hbm.at[c, pl.ds(s * CP_ROWS, CP_ROWS)],
            )

    return agg


RB = 1000


def _norms(deg_ref):
    ns = lax.rsqrt(jnp.maximum(deg_ref[0][:, 0:1], 1.0))
    nd = lax.rsqrt(jnp.maximum(deg_ref[1][:, 0:1], 1.0))
    return ns, nd


def _stage_a(deg_ref, feat_ref, g0_ref):
    ns, _ = _norms(deg_ref)
    g = feat_ref[...] * ns
    g0_ref[0] = g[:, :128]
    g0_ref[1] = g[:, 128:]


def _stage_bc(deg_ref, agg_ref, w_ref, b_ref, w2_ref, out_ref, *, last):
    ns, nd = _norms(deg_ref)
    m = jnp.dot(agg_ref[0], w_ref[:128, :], preferred_element_type=jnp.float32)
    m = m + jnp.dot(agg_ref[1], w_ref[128:, :], preferred_element_type=jnp.float32)
    h = jnp.maximum(m * nd + b_ref[...], 0.0) * ns
    if last:
        out_ref[...] = jnp.dot(h, w2_ref[...], preferred_element_type=jnp.float32)
    else:
        out_ref[0] = h[:, :128]
        out_ref[1] = h[:, 128:]


def _stage_d(deg_ref, p_ref, b_ref, out_ref):
    _, nd = _norms(deg_ref)
    a = p_ref[0] + p_ref[1]
    out_ref[...] = a[:, :D_OUT] * nd + b_ref[...]


_DEG_SPEC = pl.BlockSpec((NC, RB, DW), lambda i: (0, i, 0))


def _tc_stage_a(deg, features):
    return pl.pallas_call(
        _stage_a,
        grid=(N // RB,),
        in_specs=[_DEG_SPEC, pl.BlockSpec((RB, D_IN), lambda i: (i, 0))],
        out_specs=pl.BlockSpec((NC, RB, 128), lambda i: (0, i, 0)),
        out_shape=jax.ShapeDtypeStruct((NC, N, 128), jnp.float32),
    )(deg, features)


def _tc_stage_b(deg, agg, w, b):
    def wrapped(deg_ref, agg_ref, w_ref, b_ref, out_ref):
        _stage_bc(deg_ref, agg_ref, w_ref, b_ref, None, out_ref, last=False)

    return pl.pallas_call(
        wrapped,
        grid=(N // RB,),
        in_specs=[
            _DEG_SPEC,
            pl.BlockSpec((NC, RB, 128), lambda i: (0, i, 0)),
            pl.BlockSpec((D_HID, D_HID), lambda i: (0, 0)),
            pl.BlockSpec((1, D_HID), lambda i: (0, 0)),
        ],
        out_specs=pl.BlockSpec((NC, RB, 128), lambda i: (0, i, 0)),
        out_shape=jax.ShapeDtypeStruct((NC, N, 128), jnp.float32),
    )(deg, agg, w, b)


def _tc_stage_c(deg, agg, w, b, w2):
    body = functools.partial(_stage_bc, last=True)
    return pl.pallas_call(
        body,
        grid=(N // RB,),
        in_specs=[
            _DEG_SPEC,
            pl.BlockSpec((NC, RB, 128), lambda i: (0, i, 0)),
            pl.BlockSpec((D_HID, D_HID), lambda i: (0, 0)),
            pl.BlockSpec((1, D_HID), lambda i: (0, 0)),
            pl.BlockSpec((D_HID, 128), lambda i: (0, 0)),
        ],
        out_specs=pl.BlockSpec((RB, 128), lambda i: (i, 0)),
        out_shape=jax.ShapeDtypeStruct((N, 128), jnp.float32),
    )(deg, agg, w, b, w2)


def _tc_stage_d(deg, p, b2):
    return pl.pallas_call(
        _stage_d,
        grid=(N // RB,),
        in_specs=[
            _DEG_SPEC,
            pl.BlockSpec((NC, RB, 128), lambda i: (0, i, 0)),
            pl.BlockSpec((1, D_OUT), lambda i: (0, 0)),
        ],
        out_specs=pl.BlockSpec((RB, D_OUT), lambda i: (i, 0)),
        out_shape=jax.ShapeDtypeStruct((N, D_OUT), jnp.float32),
    )(deg, p, b2)


@jax.jit
def kernel(features, edge_index, W0, b0, W1, b1, W2, b2):
    src3d = edge_index[0].reshape(N_CHUNKS, 1, CH)
    dst3d = edge_index[1].reshape(N_CHUNKS, 1, CH)
    ei3d = edge_index.reshape(2 * N_CHUNKS, 1, CH)

    dummy = jnp.zeros((CH, 128), jnp.float32)

    deg = _make_degrees()(ei3d, dummy)
    g0 = _tc_stage_a(deg, features)
    a0 = _make_agg(128, True)(g0, src3d, dst3d, dummy)
    g1 = _tc_stage_b(deg, a0, W0, b0.reshape(1, D_HID))
    a1 = _make_agg(128, True)(g1, src3d, dst3d, dummy)
    w2p = jnp.concatenate(
        [W2, jnp.zeros((D_HID, 128 - D_OUT), jnp.float32)], axis=1
    )
    g2 = _tc_stage_c(deg, a1, W1, b1.reshape(1, D_HID), w2p)
    p2 = _make_agg(128, False)(g2, src3d, dst3d, dummy)
    out = _tc_stage_d(deg, p2, b2.reshape(1, D_OUT))
    return out
